# baseline (speedup 1.0000x reference)
import functools

import jax
import jax.numpy as jnp
from jax import lax
from jax.experimental import pallas as pl
from jax.experimental.pallas import tpu as pltpu
from jax.experimental.pallas import tpu_sc as plsc

_NC = 2
_NS = 16
_NW = _NC * _NS
_P = 16
_PK = 8
_NPAD = 102400


def _bd(w):
    return jnp.kron(jnp.eye(_PK, dtype=w.dtype), w)


def _bd_rowpad(w):
    return _bd(jnp.pad(w, ((0, _P - w.shape[0]), (0, 0))))


def _bd_colpad(w):
    return _bd(jnp.pad(w, ((0, 0), (0, _P - w.shape[1]))))


def _tile8(b):
    return jnp.tile(b, _PK)[None]


def _sc_repack(src_flat, zeros, n, d_in, out_rows):
    per_w = n // _NW
    C = 2000 if per_w % 2000 == 0 else per_w
    n_chunks = per_w // C
    nb = -(-C // 16)
    tail = out_rows - n
    mesh = plsc.VectorSubcoreMesh(core_axis_name="c", subcore_axis_name="s")

    @functools.partial(
        pl.kernel,
        mesh=mesh,
        out_type=jax.ShapeDtypeStruct((out_rows, _P), jnp.float32),
        scratch_types=[
            pltpu.VMEM((C * d_in,), jnp.float32),
            pltpu.VMEM((C, _P), jnp.float32),
        ],
        compiler_params=pltpu.CompilerParams(
            use_tc_tiling_on_sc=False, needs_layout_passes=False
        ),
    )
    def k(src_h, zeros_h, out_h, bufs, buf16):
        wid = lax.axis_index("s") * _NC + lax.axis_index("c")
        base = wid * per_w
        pltpu.sync_copy(zeros_h.at[pl.ds(0, C)], buf16)
        if tail:
            @pl.when(wid == 0)
            def _():
                pltpu.sync_copy(
                    zeros_h.at[pl.ds(0, tail)], out_h.at[pl.ds(n, tail)]
                )

        def chunk(i, carry):
            row0 = base + i * C
            for kf in range(d_in):
                pltpu.sync_copy(
                    src_h.at[pl.ds(kf * n + row0, C)],
                    bufs.at[pl.ds(kf * C, C)],
                )

            def batch(kk, c2):
                iota = lax.iota(jnp.int32, 16)
                e = jnp.minimum(kk * 16 + iota, C - 1)
                for j in range(d_in):
                    v = plsc.load_gather(bufs, [j * C + e])
                    plsc.store_scatter(
                        buf16, [e, jnp.full((16,), j, jnp.int32)], v
                    )
                return c2

            lax.fori_loop(0, nb, batch, 0)
            pltpu.sync_copy(buf16, out_h.at[pl.ds(row0, C)])
            return carry

        lax.fori_loop(0, n_chunks, chunk, 0)

    return k(src_flat, zeros)


def _sc_gather2(table, idx_a, idx_b):
    E = idx_a.shape[0]
    per_w = E // _NW
    C = 2000
    n_chunks = per_w // C
    mesh = plsc.VectorSubcoreMesh(core_axis_name="c", subcore_axis_name="s")

    @functools.partial(
        pl.kernel,
        mesh=mesh,
        out_type=(
            jax.ShapeDtypeStruct((E, _P), jnp.float32),
            jax.ShapeDtypeStruct((E, _P), jnp.float32),
        ),
        scratch_types=[
            pltpu.VMEM((C,), jnp.int32),
            pltpu.VMEM((C,), jnp.int32),
            pltpu.VMEM((C, _P), jnp.float32),
            pltpu.VMEM((C, _P), jnp.float32),
            pltpu.SemaphoreType.DMA,
            pltpu.SemaphoreType.DMA,
        ],
        compiler_params=pltpu.CompilerParams(use_tc_tiling_on_sc=False),
    )
    def k(table_h, ia_h, ib_h, oa, ob, ia_v, ib_v, ra_v, rb_v, sa, sb):
        table = table_h
        wid = lax.axis_index("s") * _NC + lax.axis_index("c")
        base = pl.multiple_of(wid * per_w, 8)

        def body(i, carry):
            off = pl.multiple_of(base + i * C, 8)
            pltpu.sync_copy(ia_h.at[pl.ds(off, C)], ia_v)
            pltpu.sync_copy(ib_h.at[pl.ds(off, C)], ib_v)
            ca = pltpu.async_copy(table.at[ia_v], ra_v, sa)
            cb = pltpu.async_copy(table.at[ib_v], rb_v, sb)
            ca.wait()
            cb.wait()
            pltpu.sync_copy(ra_v, oa.at[pl.ds(off, C)])
            pltpu.sync_copy(rb_v, ob.at[pl.ds(off, C)])
            return carry

        lax.fori_loop(0, n_chunks, body, 0)

    return k(table, idx_a, idx_b)


def _sc_scatter_add(vals, idx, zeros):
    E = idx.shape[0]
    per_core = E // _NC
    per_sub = per_core // _NS
    C = 1000
    n_chunks = per_sub // C
    rows = _NPAD // _NS
    mesh = plsc.VectorSubcoreMesh(core_axis_name="c", subcore_axis_name="s")

    @functools.partial(
        pl.kernel,
        mesh=mesh,
        out_type=jax.ShapeDtypeStruct((_NC * _NPAD, _P), jnp.float32),
        scratch_types=[
            pltpu.VMEM((C,), jnp.int32),
            pltpu.VMEM((C, _P), jnp.float32),
            pltpu.VMEM_SHARED((_NPAD, _P), jnp.float32),
            pltpu.SemaphoreType.DMA,
        ],
        compiler_params=pltpu.CompilerParams(use_tc_tiling_on_sc=False),
    )
    def k(vals_h, idx_h, zeros_h, out_h, idx_v, vals_v, acc, sem):
        vals = vals_h
        zeros = zeros_h
        out = out_h
        cid = lax.axis_index("c")
        sid = lax.axis_index("s")
        pltpu.sync_copy(
            zeros.at[pl.ds(sid * rows, rows)], acc.at[pl.ds(sid * rows, rows)]
        )
        plsc.subcore_barrier()
        base = cid * per_core + sid * per_sub

        def body(i, carry):
            off = pl.multiple_of(base + i * C, 8)
            pltpu.sync_copy(idx_h.at[pl.ds(off, C)], idx_v)
            pltpu.sync_copy(vals.at[pl.ds(off, C)], vals_v)
            pltpu.sync_copy(vals_v, acc.at[idx_v], add=True)
            return carry

        lax.fori_loop(0, n_chunks, body, 0)
        plsc.subcore_barrier()
        pltpu.sync_copy(
            acc.at[pl.ds(sid * rows, rows)],
            out.at[pl.ds(cid * _NPAD + sid * rows, rows)],
        )

    return k(vals, idx, zeros)


def _edge_mlp3(xi8, xj8, ea8, Wi, Wj, We, b0, W1, b1, W2, b2, final=False):
    E8 = xi8.shape[0]
    H8 = W1.shape[0]
    do8 = W2.shape[1]
    BE8 = 1000
    grid = E8 // BE8

    def body(xi_r, xj_r, ea_r, wi, wj, we, b0r, w1, b1r, w2, b2r, o_r):
        bf = jnp.bfloat16
        h = (
            jnp.dot(xi_r[...].astype(bf), wi[...].astype(bf),
                    preferred_element_type=jnp.float32)
            + jnp.dot(xj_r[...].astype(bf), wj[...].astype(bf),
                      preferred_element_type=jnp.float32)
            + jnp.dot(ea_r[...].astype(bf), we[...].astype(bf),
                      preferred_element_type=jnp.float32)
            + b0r[...]
        )
        h = jnp.maximum(h, 0.0)
        h = jnp.maximum(
            jnp.dot(h.astype(bf), w1[...].astype(bf),
                    preferred_element_type=jnp.float32) + b1r[...], 0.0
        )
        o = jnp.dot(h.astype(bf), w2[...].astype(bf),
                    preferred_element_type=jnp.float32) + b2r[...]
        if final:
            o = jax.nn.sigmoid(o)
        o_r[...] = o

    wspec = lambda s: pl.BlockSpec(s, lambda i: (0, 0))
    if final:
        out_shape = jax.ShapeDtypeStruct((E8, _PK), jnp.float32)
        out_spec = pl.BlockSpec((BE8, _PK), lambda i: (i, 0))
    else:
        out_shape = jax.ShapeDtypeStruct((E8, 128), jnp.float32)
        out_spec = pl.BlockSpec((BE8, 128), lambda i: (i, 0))
    return pl.pallas_call(
        body,
        grid=(grid,),
        in_specs=[
            pl.BlockSpec((BE8, 128), lambda i: (i, 0)),
            pl.BlockSpec((BE8, 128), lambda i: (i, 0)),
            pl.BlockSpec((BE8, 128), lambda i: (i, 0)),
            wspec((128, H8)),
            wspec((128, H8)),
            wspec((128, H8)),
            wspec((1, H8)),
            wspec((H8, H8)),
            wspec((1, H8)),
            wspec((H8, do8)),
            wspec((1, do8)),
        ],
        out_specs=out_spec,
        out_shape=out_shape,
    )(xi8, xj8, ea8, Wi, Wj, We, b0, W1, b1, W2, b2)


def _node_mlp3(x8, p8, Wx, Wa, b0, W1, b1, W2, b2):
    N8 = x8.shape[0]
    H8 = W1.shape[0]
    BN8 = 800
    grid = N8 // BN8

    def body(x_r, p0_r, p1_r, wx, wa, b0r, w1, b1r, w2, b2r, o_r):
        bf = jnp.bfloat16
        aggr = p0_r[...] + p1_r[...]
        h = (
            jnp.dot(x_r[...].astype(bf), wx[...].astype(bf),
                    preferred_element_type=jnp.float32)
            + jnp.dot(aggr.astype(bf), wa[...].astype(bf),
                      preferred_element_type=jnp.float32)
            + b0r[...]
        )
        h = jnp.maximum(h, 0.0)
        h = jnp.maximum(
            jnp.dot(h.astype(bf), w1[...].astype(bf),
                    preferred_element_type=jnp.float32) + b1r[...], 0.0
        )
        o_r[...] = jnp.dot(h.astype(bf), w2[...].astype(bf),
                           preferred_element_type=jnp.float32) + b2r[...]

    wspec = lambda s: pl.BlockSpec(s, lambda i: (0, 0))
    nblocks = N8 // BN8
    return pl.pallas_call(
        body,
        grid=(grid,),
        in_specs=[
            pl.BlockSpec((BN8, 128), lambda i: (i, 0)),
            pl.BlockSpec((BN8, 128), lambda i: (i, 0)),
            pl.BlockSpec((BN8, 128), lambda i: (nblocks + i, 0)),
            wspec((128, H8)),
            wspec((128, H8)),
            wspec((1, H8)),
            wspec((H8, H8)),
            wspec((1, H8)),
            wspec((H8, 128)),
            wspec((1, 128)),
        ],
        out_specs=pl.BlockSpec((BN8, 128), lambda i: (i, 0)),
        out_shape=jax.ShapeDtypeStruct((N8, 128), jnp.float32),
    )(x8, p8, p8, Wx, Wa, b0, W1, b1, W2, b2)


def kernel(x, edge_index, edge_attr, params):
    src = edge_index[0].astype(jnp.int32)
    dst = edge_index[1].astype(jnp.int32)
    N = x.shape[0]
    E = edge_attr.shape[0]
    E8 = E // _PK
    n8 = _NPAD // _PK

    zeros = jnp.zeros((_NPAD, _P), jnp.float32)
    n_work = -(-N // (8 * _NW)) * 8 * _NW
    xflat = jnp.pad(x.T, ((0, 0), (0, n_work - N))).reshape(-1)
    x8 = _sc_repack(xflat, zeros, n_work, 3, _NPAD).reshape(n8, 128)
    ea8 = _sc_repack(edge_attr.T.reshape(-1), zeros, E, 3, E).reshape(E8, 128)
    d = 3

    for name in ("IN1", "IN2", "IN3"):
        layer = params[name]
        (W0, b0), (W1, b1), (W2, b2) = layer["R1"]
        xi, xj = _sc_gather2(x8.reshape(_NPAD, _P), dst, src)
        e8 = _edge_mlp3(
            xi.reshape(E8, 128), xj.reshape(E8, 128), ea8,
            _bd_rowpad(W0[:d]), _bd_rowpad(W0[d : 2 * d]), _bd_rowpad(W0[2 * d :]),
            _tile8(b0), _bd(W1), _tile8(b1),
            _bd_colpad(W2), _tile8(jnp.pad(b2, (0, _P - b2.shape[0]))),
        )
        p = _sc_scatter_add(e8.reshape(E, _P), dst, zeros)
        (V0, c0), (V1, c1), (V2, c2) = layer["O"]
        x8 = _node_mlp3(
            x8, p.reshape(_NC * n8, 128),
            _bd_rowpad(V0[:d]), _bd_rowpad(V0[d:]),
            _tile8(c0), _bd(V1), _tile8(c1),
            _bd_colpad(V2), _tile8(jnp.pad(c2, (0, _P - c2.shape[0]))),
        )
        ea8 = e8
        d = W2.shape[1]

    (W0, b0), (W1, b1), (W2, b2) = params["R2"]
    xi, xj = _sc_gather2(x8.reshape(_NPAD, _P), dst, src)
    out8 = _edge_mlp3(
        xi.reshape(E8, 128), xj.reshape(E8, 128), ea8,
        _bd_rowpad(W0[:d]), _bd_rowpad(W0[d : 2 * d]), _bd_rowpad(W0[2 * d :]),
        _tile8(b0), _bd(W1), _tile8(b1), _bd(W2), _tile8(b2),
        final=True,
    )
    return out8.reshape(E, 1)


# speedup vs baseline: 1.0092x; 1.0092x over previous
import functools

import jax
import jax.numpy as jnp
from jax import lax
from jax.experimental import pallas as pl
from jax.experimental.pallas import tpu as pltpu
from jax.experimental.pallas import tpu_sc as plsc

_NC = 2
_NS = 16
_NW = _NC * _NS
_P = 16
_PK = 8
_NPAD = 102400


def _bd(w):
    return jnp.kron(jnp.eye(_PK, dtype=w.dtype), w)


def _bd_rowpad(w):
    return _bd(jnp.pad(w, ((0, _P - w.shape[0]), (0, 0))))


def _bd_colpad(w):
    return _bd(jnp.pad(w, ((0, 0), (0, _P - w.shape[1]))))


def _tile8(b):
    return jnp.tile(b, _PK)[None]


def _sc_repack(src_t, zeros, n, d_in, out_rows):
    per_w = n // _NW
    C = 2000 if per_w % 2000 == 0 else per_w
    n_chunks = per_w // C
    nb = -(-C // 16)
    tail = out_rows - n
    mesh = plsc.VectorSubcoreMesh(core_axis_name="c", subcore_axis_name="s")

    @functools.partial(
        pl.kernel,
        mesh=mesh,
        out_type=jax.ShapeDtypeStruct((out_rows, _P), jnp.float32),
        scratch_types=[
            pltpu.VMEM((C * d_in,), jnp.float32),
            pltpu.VMEM((C, _P), jnp.float32),
        ],
        compiler_params=pltpu.CompilerParams(
            use_tc_tiling_on_sc=False, needs_layout_passes=False
        ),
    )
    def k(src_h, zeros_h, out_h, bufs, buf16):
        wid = lax.axis_index("s") * _NC + lax.axis_index("c")
        base = wid * per_w
        pltpu.sync_copy(zeros_h.at[pl.ds(0, C)], buf16)
        if tail:
            @pl.when(wid == 0)
            def _():
                pltpu.sync_copy(
                    zeros_h.at[pl.ds(0, tail)], out_h.at[pl.ds(n, tail)]
                )

        def chunk(i, carry):
            row0 = base + i * C
            for kf in range(d_in):
                pltpu.sync_copy(
                    src_h.at[kf, pl.ds(row0, C)],
                    bufs.at[pl.ds(kf * C, C)],
                )

            def batch(kk, c2):
                iota = lax.iota(jnp.int32, 16)
                e = jnp.minimum(kk * 16 + iota, C - 1)
                for j in range(d_in):
                    v = plsc.load_gather(bufs, [j * C + e])
                    plsc.store_scatter(
                        buf16, [e, jnp.full((16,), j, jnp.int32)], v
                    )
                return c2

            lax.fori_loop(0, nb, batch, 0)
            pltpu.sync_copy(buf16, out_h.at[pl.ds(row0, C)])
            return carry

        lax.fori_loop(0, n_chunks, chunk, 0)

    return k(src_t, zeros)


def _sc_gather2(table, idx_a, idx_b):
    E = idx_a.shape[0]
    per_w = E // _NW
    C = 2000
    n_chunks = per_w // C
    mesh = plsc.VectorSubcoreMesh(core_axis_name="c", subcore_axis_name="s")

    @functools.partial(
        pl.kernel,
        mesh=mesh,
        out_type=(
            jax.ShapeDtypeStruct((E, _P), jnp.float32),
            jax.ShapeDtypeStruct((E, _P), jnp.float32),
        ),
        scratch_types=[
            pltpu.VMEM((C,), jnp.int32),
            pltpu.VMEM((C,), jnp.int32),
            pltpu.VMEM((C, _P), jnp.float32),
            pltpu.VMEM((C, _P), jnp.float32),
            pltpu.SemaphoreType.DMA,
            pltpu.SemaphoreType.DMA,
        ],
        compiler_params=pltpu.CompilerParams(use_tc_tiling_on_sc=False),
    )
    def k(table_h, ia_h, ib_h, oa, ob, ia_v, ib_v, ra_v, rb_v, sa, sb):
        table = table_h
        wid = lax.axis_index("s") * _NC + lax.axis_index("c")
        base = pl.multiple_of(wid * per_w, 8)

        def body(i, carry):
            off = pl.multiple_of(base + i * C, 8)
            pltpu.sync_copy(ia_h.at[pl.ds(off, C)], ia_v)
            pltpu.sync_copy(ib_h.at[pl.ds(off, C)], ib_v)
            ca = pltpu.async_copy(table.at[ia_v], ra_v, sa)
            cb = pltpu.async_copy(table.at[ib_v], rb_v, sb)
            ca.wait()
            cb.wait()
            pltpu.sync_copy(ra_v, oa.at[pl.ds(off, C)])
            pltpu.sync_copy(rb_v, ob.at[pl.ds(off, C)])
            return carry

        lax.fori_loop(0, n_chunks, body, 0)

    return k(table, idx_a, idx_b)


def _sc_scatter_add(vals, idx, zeros):
    E = idx.shape[0]
    per_core = E // _NC
    per_sub = per_core // _NS
    C = 1000
    n_chunks = per_sub // C
    rows = _NPAD // _NS
    mesh = plsc.VectorSubcoreMesh(core_axis_name="c", subcore_axis_name="s")

    @functools.partial(
        pl.kernel,
        mesh=mesh,
        out_type=jax.ShapeDtypeStruct((_NC * _NPAD, _P), jnp.float32),
        scratch_types=[
            pltpu.VMEM((C,), jnp.int32),
            pltpu.VMEM((C, _P), jnp.float32),
            pltpu.VMEM_SHARED((_NPAD, _P), jnp.float32),
            pltpu.SemaphoreType.DMA,
        ],
        compiler_params=pltpu.CompilerParams(use_tc_tiling_on_sc=False),
    )
    def k(vals_h, idx_h, zeros_h, out_h, idx_v, vals_v, acc, sem):
        vals = vals_h
        zeros = zeros_h
        out = out_h
        cid = lax.axis_index("c")
        sid = lax.axis_index("s")
        pltpu.sync_copy(
            zeros.at[pl.ds(sid * rows, rows)], acc.at[pl.ds(sid * rows, rows)]
        )
        plsc.subcore_barrier()
        base = cid * per_core + sid * per_sub

        def body(i, carry):
            off = pl.multiple_of(base + i * C, 8)
            pltpu.sync_copy(idx_h.at[pl.ds(off, C)], idx_v)
            pltpu.sync_copy(vals.at[pl.ds(off, C)], vals_v)
            pltpu.sync_copy(vals_v, acc.at[idx_v], add=True)
            return carry

        lax.fori_loop(0, n_chunks, body, 0)
        plsc.subcore_barrier()
        pltpu.sync_copy(
            acc.at[pl.ds(sid * rows, rows)],
            out.at[pl.ds(cid * _NPAD + sid * rows, rows)],
        )

    return k(vals, idx, zeros)


def _edge_mlp3(xi8, xj8, ea8, Wi, Wj, We, b0, W1, b1, W2, b2, final=False):
    E8 = xi8.shape[0]
    H8 = W1.shape[0]
    do8 = W2.shape[1]
    BE8 = 1000
    grid = E8 // BE8

    def body(xi_r, xj_r, ea_r, wi, wj, we, b0r, w1, b1r, w2, b2r, o_r):
        h = (
            jnp.dot(xi_r[...], wi[...], preferred_element_type=jnp.float32)
            + jnp.dot(xj_r[...], wj[...], preferred_element_type=jnp.float32)
            + jnp.dot(ea_r[...], we[...], preferred_element_type=jnp.float32)
            + b0r[...]
        )
        h = jnp.maximum(h, 0.0)
        h = jnp.maximum(
            jnp.dot(h, w1[...], preferred_element_type=jnp.float32) + b1r[...], 0.0
        )
        o = jnp.dot(h, w2[...], preferred_element_type=jnp.float32) + b2r[...]
        if final:
            o = jax.nn.sigmoid(o)
        o_r[...] = o

    wspec = lambda s: pl.BlockSpec(s, lambda i: (0, 0))
    if final:
        out_shape = jax.ShapeDtypeStruct((E8, _PK), jnp.float32)
        out_spec = pl.BlockSpec((BE8, _PK), lambda i: (i, 0))
    else:
        out_shape = jax.ShapeDtypeStruct((E8, 128), jnp.float32)
        out_spec = pl.BlockSpec((BE8, 128), lambda i: (i, 0))
    return pl.pallas_call(
        body,
        grid=(grid,),
        in_specs=[
            pl.BlockSpec((BE8, 128), lambda i: (i, 0)),
            pl.BlockSpec((BE8, 128), lambda i: (i, 0)),
            pl.BlockSpec((BE8, 128), lambda i: (i, 0)),
            wspec((128, H8)),
            wspec((128, H8)),
            wspec((128, H8)),
            wspec((1, H8)),
            wspec((H8, H8)),
            wspec((1, H8)),
            wspec((H8, do8)),
            wspec((1, do8)),
        ],
        out_specs=out_spec,
        out_shape=out_shape,
    )(xi8, xj8, ea8, Wi, Wj, We, b0, W1, b1, W2, b2)


def _node_mlp3(x8, p8, Wx, Wa, b0, W1, b1, W2, b2):
    N8 = x8.shape[0]
    H8 = W1.shape[0]
    BN8 = 800
    grid = N8 // BN8

    def body(x_r, p0_r, p1_r, wx, wa, b0r, w1, b1r, w2, b2r, o_r):
        aggr = p0_r[...] + p1_r[...]
        h = (
            jnp.dot(x_r[...], wx[...], preferred_element_type=jnp.float32)
            + jnp.dot(aggr, wa[...], preferred_element_type=jnp.float32)
            + b0r[...]
        )
        h = jnp.maximum(h, 0.0)
        h = jnp.maximum(
            jnp.dot(h, w1[...], preferred_element_type=jnp.float32) + b1r[...], 0.0
        )
        o_r[...] = jnp.dot(h, w2[...], preferred_element_type=jnp.float32) + b2r[...]

    wspec = lambda s: pl.BlockSpec(s, lambda i: (0, 0))
    nblocks = N8 // BN8
    return pl.pallas_call(
        body,
        grid=(grid,),
        in_specs=[
            pl.BlockSpec((BN8, 128), lambda i: (i, 0)),
            pl.BlockSpec((BN8, 128), lambda i: (i, 0)),
            pl.BlockSpec((BN8, 128), lambda i: (nblocks + i, 0)),
            wspec((128, H8)),
            wspec((128, H8)),
            wspec((1, H8)),
            wspec((H8, H8)),
            wspec((1, H8)),
            wspec((H8, 128)),
            wspec((1, 128)),
        ],
        out_specs=pl.BlockSpec((BN8, 128), lambda i: (i, 0)),
        out_shape=jax.ShapeDtypeStruct((N8, 128), jnp.float32),
    )(x8, p8, p8, Wx, Wa, b0, W1, b1, W2, b2)


def kernel(x, edge_index, edge_attr, params):
    src = edge_index[0].astype(jnp.int32)
    dst = edge_index[1].astype(jnp.int32)
    N = x.shape[0]
    E = edge_attr.shape[0]
    E8 = E // _PK
    n8 = _NPAD // _PK

    zeros = jnp.zeros((_NPAD, _P), jnp.float32)
    n_work = -(-N // (8 * _NW)) * 8 * _NW
    xt = jnp.pad(x.T, ((0, 0), (0, n_work - N)))
    x8 = _sc_repack(xt, zeros, n_work, 3, _NPAD).reshape(n8, 128)
    ea8 = _sc_repack(edge_attr.T, zeros, E, 3, E).reshape(E8, 128)
    d = 3

    for name in ("IN1", "IN2", "IN3"):
        layer = params[name]
        (W0, b0), (W1, b1), (W2, b2) = layer["R1"]
        xi, xj = _sc_gather2(x8.reshape(_NPAD, _P), dst, src)
        e8 = _edge_mlp3(
            xi.reshape(E8, 128), xj.reshape(E8, 128), ea8,
            _bd_rowpad(W0[:d]), _bd_rowpad(W0[d : 2 * d]), _bd_rowpad(W0[2 * d :]),
            _tile8(b0), _bd(W1), _tile8(b1),
            _bd_colpad(W2), _tile8(jnp.pad(b2, (0, _P - b2.shape[0]))),
        )
        p = _sc_scatter_add(e8.reshape(E, _P), dst, zeros)
        (V0, c0), (V1, c1), (V2, c2) = layer["O"]
        x8 = _node_mlp3(
            x8, p.reshape(_NC * n8, 128),
            _bd_rowpad(V0[:d]), _bd_rowpad(V0[d:]),
            _tile8(c0), _bd(V1), _tile8(c1),
            _bd_colpad(V2), _tile8(jnp.pad(c2, (0, _P - c2.shape[0]))),
        )
        ea8 = e8
        d = W2.shape[1]

    (W0, b0), (W1, b1), (W2, b2) = params["R2"]
    xi, xj = _sc_gather2(x8.reshape(_NPAD, _P), dst, src)
    out8 = _edge_mlp3(
        xi.reshape(E8, 128), xj.reshape(E8, 128), ea8,
        _bd_rowpad(W0[:d]), _bd_rowpad(W0[d : 2 * d]), _bd_rowpad(W0[2 * d :]),
        _tile8(b0), _bd(W1), _tile8(b1), _bd(W2), _tile8(b2),
        final=True,
    )
    return out8.reshape(E, 1)


# speedup vs baseline: 1.1595x; 1.1490x over previous
import functools

import jax
import jax.numpy as jnp
from jax import lax
from jax.experimental import pallas as pl
from jax.experimental.pallas import tpu as pltpu
from jax.experimental.pallas import tpu_sc as plsc

_NC = 2
_NS = 16
_NW = _NC * _NS
_P = 16
_PK = 8
_NPAD = 102400


def _bd(w):
    return jnp.kron(jnp.eye(_PK, dtype=w.dtype), w)


def _bd_rowpad(w):
    return _bd(jnp.pad(w, ((0, _P - w.shape[0]), (0, 0))))


def _bd_colpad(w):
    return _bd(jnp.pad(w, ((0, 0), (0, _P - w.shape[1]))))


def _tile8(b):
    return jnp.tile(b, _PK)[None]


def _sc_repack(src_t, zeros, n, d_in, out_rows):
    per_w = n // _NW
    C = 2000 if per_w % 2000 == 0 else per_w
    n_chunks = per_w // C
    nb = -(-C // 16)
    tail = out_rows - n
    mesh = plsc.VectorSubcoreMesh(core_axis_name="c", subcore_axis_name="s")

    @functools.partial(
        pl.kernel,
        mesh=mesh,
        out_type=jax.ShapeDtypeStruct((out_rows, _P), jnp.float32),
        scratch_types=[
            pltpu.VMEM((C * d_in,), jnp.float32),
            pltpu.VMEM((C, _P), jnp.float32),
        ],
        compiler_params=pltpu.CompilerParams(
            use_tc_tiling_on_sc=False, needs_layout_passes=False
        ),
    )
    def k(src_h, zeros_h, out_h, bufs, buf16):
        wid = lax.axis_index("s") * _NC + lax.axis_index("c")
        base = wid * per_w
        pltpu.sync_copy(zeros_h.at[pl.ds(0, C)], buf16)
        if tail:
            @pl.when(wid == 0)
            def _():
                pltpu.sync_copy(
                    zeros_h.at[pl.ds(0, tail)], out_h.at[pl.ds(n, tail)]
                )

        def chunk(i, carry):
            row0 = base + i * C
            for kf in range(d_in):
                pltpu.sync_copy(
                    src_h.at[kf, pl.ds(row0, C)],
                    bufs.at[pl.ds(kf * C, C)],
                )

            def batch(kk, c2):
                iota = lax.iota(jnp.int32, 16)
                e = jnp.minimum(kk * 16 + iota, C - 1)
                for j in range(d_in):
                    v = plsc.load_gather(bufs, [j * C + e])
                    plsc.store_scatter(
                        buf16, [e, jnp.full((16,), j, jnp.int32)], v
                    )
                return c2

            lax.fori_loop(0, nb, batch, 0)
            pltpu.sync_copy(buf16, out_h.at[pl.ds(row0, C)])
            return carry

        lax.fori_loop(0, n_chunks, chunk, 0)

    return k(src_t, zeros)


def _sc_gather2(table, idx_a, idx_b):
    E = idx_a.shape[0]
    per_w = E // _NW
    C = 2000 if per_w % 2000 == 0 else 1000
    n_chunks = per_w // C
    mesh = plsc.VectorSubcoreMesh(core_axis_name="c", subcore_axis_name="s")

    @functools.partial(
        pl.kernel,
        mesh=mesh,
        out_type=(
            jax.ShapeDtypeStruct((E, _P), jnp.float32),
            jax.ShapeDtypeStruct((E, _P), jnp.float32),
        ),
        scratch_types=[
            pltpu.VMEM((C,), jnp.int32),
            pltpu.VMEM((C,), jnp.int32),
            pltpu.VMEM((C, _P), jnp.float32),
            pltpu.VMEM((C, _P), jnp.float32),
            pltpu.SemaphoreType.DMA,
            pltpu.SemaphoreType.DMA,
        ],
        compiler_params=pltpu.CompilerParams(use_tc_tiling_on_sc=False),
    )
    def k(table_h, ia_h, ib_h, oa, ob, ia_v, ib_v, ra_v, rb_v, sa, sb):
        table = table_h
        wid = lax.axis_index("s") * _NC + lax.axis_index("c")
        base = pl.multiple_of(wid * per_w, 8)

        def body(i, carry):
            off = pl.multiple_of(base + i * C, 8)
            pltpu.sync_copy(ia_h.at[pl.ds(off, C)], ia_v)
            pltpu.sync_copy(ib_h.at[pl.ds(off, C)], ib_v)
            ca = pltpu.async_copy(table.at[ia_v], ra_v, sa)
            cb = pltpu.async_copy(table.at[ib_v], rb_v, sb)
            ca.wait()
            cb.wait()
            pltpu.sync_copy(ra_v, oa.at[pl.ds(off, C)])
            pltpu.sync_copy(rb_v, ob.at[pl.ds(off, C)])
            return carry

        lax.fori_loop(0, n_chunks, body, 0)

    return k(table, idx_a, idx_b)


def _sc_scatter_add(vals, idx, zeros):
    E = idx.shape[0]
    per_core = E // _NC
    per_sub = per_core // _NS
    C = 1000
    n_chunks = per_sub // C
    rows = _NPAD // _NS
    mesh = plsc.VectorSubcoreMesh(core_axis_name="c", subcore_axis_name="s")

    @functools.partial(
        pl.kernel,
        mesh=mesh,
        out_type=jax.ShapeDtypeStruct((_NC * _NPAD, _P), jnp.float32),
        scratch_types=[
            pltpu.VMEM((C,), jnp.int32),
            pltpu.VMEM((C, _P), jnp.float32),
            pltpu.VMEM_SHARED((_NPAD, _P), jnp.float32),
            pltpu.SemaphoreType.DMA,
        ],
        compiler_params=pltpu.CompilerParams(use_tc_tiling_on_sc=False),
    )
    def k(vals_h, idx_h, zeros_h, out_h, idx_v, vals_v, acc, sem):
        vals = vals_h
        zeros = zeros_h
        out = out_h
        cid = lax.axis_index("c")
        sid = lax.axis_index("s")
        pltpu.sync_copy(
            zeros.at[pl.ds(sid * rows, rows)], acc.at[pl.ds(sid * rows, rows)]
        )
        plsc.subcore_barrier()
        base = cid * per_core + sid * per_sub

        def body(i, carry):
            off = pl.multiple_of(base + i * C, 8)
            pltpu.sync_copy(idx_h.at[pl.ds(off, C)], idx_v)
            pltpu.sync_copy(vals.at[pl.ds(off, C)], vals_v)
            pltpu.sync_copy(vals_v, acc.at[idx_v], add=True)
            return carry

        lax.fori_loop(0, n_chunks, body, 0)
        plsc.subcore_barrier()
        pltpu.sync_copy(
            acc.at[pl.ds(sid * rows, rows)],
            out.at[pl.ds(cid * _NPAD + sid * rows, rows)],
        )

    return k(vals, idx, zeros)


def _edge_mlp3(xi8, xj8, ea8, Wi, Wj, We, b0, W1, b1, W2, b2, final=False,
               ea_off=0, out_off=0, out_rows=None, out_alias=None):
    E8 = xi8.shape[0]
    H8 = W1.shape[0]
    do8 = W2.shape[1]
    BE8 = 1000
    grid = E8 // BE8
    if out_rows is None:
        out_rows = E8

    def body(xi_r, xj_r, ea_r, wi, wj, we, b0r, w1, b1r, w2, b2r, *rest):
        o_r = rest[-1]
        h = (
            jnp.dot(xi_r[...], wi[...], preferred_element_type=jnp.float32)
            + jnp.dot(xj_r[...], wj[...], preferred_element_type=jnp.float32)
            + jnp.dot(ea_r[...], we[...], preferred_element_type=jnp.float32)
            + b0r[...]
        )
        h = jnp.maximum(h, 0.0)
        h = jnp.maximum(
            jnp.dot(h, w1[...], preferred_element_type=jnp.float32) + b1r[...], 0.0
        )
        o = jnp.dot(h, w2[...], preferred_element_type=jnp.float32) + b2r[...]
        if final:
            o = jax.nn.sigmoid(o)
        o_r[...] = o

    wspec = lambda s: pl.BlockSpec(s, lambda i: (0, 0))
    do = _PK if final else 128
    out_shape = jax.ShapeDtypeStruct((out_rows, do), jnp.float32)
    out_spec = pl.BlockSpec((BE8, do), lambda i: (out_off + i, 0))
    args = [xi8, xj8, ea8, Wi, Wj, We, b0, W1, b1, W2, b2]
    in_specs = [
        pl.BlockSpec((BE8, 128), lambda i: (i, 0)),
        pl.BlockSpec((BE8, 128), lambda i: (i, 0)),
        pl.BlockSpec((BE8, 128), lambda i: (ea_off + i, 0)),
        wspec((128, H8)),
        wspec((128, H8)),
        wspec((128, H8)),
        wspec((1, H8)),
        wspec((H8, H8)),
        wspec((1, H8)),
        wspec((H8, do8)),
        wspec((1, do8)),
    ]
    aliases = {}
    if out_alias is not None:
        args.append(out_alias)
        in_specs.append(pl.BlockSpec((BE8, do), lambda i: (out_off + i, 0)))
        aliases = {len(args) - 1: 0}
    return pl.pallas_call(
        body,
        grid=(grid,),
        in_specs=in_specs,
        out_specs=out_spec,
        out_shape=out_shape,
        input_output_aliases=aliases,
    )(*args)


def _node_mlp3(x8, pa8, pb8, Wx, Wa, b0, W1, b1, W2, b2):
    N8 = x8.shape[0]
    H8 = W1.shape[0]
    BN8 = 800
    grid = N8 // BN8

    def body(x_r, p0_r, p1_r, p2_r, p3_r, wx, wa, b0r, w1, b1r, w2, b2r, o_r):
        aggr = p0_r[...] + p1_r[...] + p2_r[...] + p3_r[...]
        h = (
            jnp.dot(x_r[...], wx[...], preferred_element_type=jnp.float32)
            + jnp.dot(aggr, wa[...], preferred_element_type=jnp.float32)
            + b0r[...]
        )
        h = jnp.maximum(h, 0.0)
        h = jnp.maximum(
            jnp.dot(h, w1[...], preferred_element_type=jnp.float32) + b1r[...], 0.0
        )
        o_r[...] = jnp.dot(h, w2[...], preferred_element_type=jnp.float32) + b2r[...]

    wspec = lambda s: pl.BlockSpec(s, lambda i: (0, 0))
    nblocks = N8 // BN8
    return pl.pallas_call(
        body,
        grid=(grid,),
        in_specs=[
            pl.BlockSpec((BN8, 128), lambda i: (i, 0)),
            pl.BlockSpec((BN8, 128), lambda i: (i, 0)),
            pl.BlockSpec((BN8, 128), lambda i: (nblocks + i, 0)),
            pl.BlockSpec((BN8, 128), lambda i: (i, 0)),
            pl.BlockSpec((BN8, 128), lambda i: (nblocks + i, 0)),
            wspec((128, H8)),
            wspec((128, H8)),
            wspec((1, H8)),
            wspec((H8, H8)),
            wspec((1, H8)),
            wspec((H8, 128)),
            wspec((1, 128)),
        ],
        out_specs=pl.BlockSpec((BN8, 128), lambda i: (i, 0)),
        out_shape=jax.ShapeDtypeStruct((N8, 128), jnp.float32),
    )(x8, pa8, pa8, pb8, pb8, Wx, Wa, b0, W1, b1, W2, b2)


def kernel(x, edge_index, edge_attr, params):
    src = edge_index[0].astype(jnp.int32)
    dst = edge_index[1].astype(jnp.int32)
    N = x.shape[0]
    E = edge_attr.shape[0]
    E8 = E // _PK
    n8 = _NPAD // _PK

    zeros = jnp.zeros((_NPAD, _P), jnp.float32)
    n_work = -(-N // (8 * _NW)) * 8 * _NW
    xt = jnp.pad(x.T, ((0, 0), (0, n_work - N)))
    x8 = _sc_repack(xt, zeros, n_work, 3, _NPAD).reshape(n8, 128)
    ea8 = _sc_repack(edge_attr.T, zeros, E, 3, E).reshape(E8, 128)
    d = 3

    Eh = E // 2
    E8h = E8 // 2
    hoff = E8h // 1000
    srcA, srcB = src[:Eh], src[Eh:]
    dstA, dstB = dst[:Eh], dst[Eh:]
    eaA, eaB = ea8, ea8
    ea_offs = (0, hoff)

    for name in ("IN1", "IN2", "IN3"):
        layer = params[name]
        (W0, b0), (W1, b1), (W2, b2) = layer["R1"]
        x16 = x8.reshape(_NPAD, _P)
        xiA, xjA = _sc_gather2(x16, dstA, srcA)
        xiB, xjB = _sc_gather2(x16, dstB, srcB)
        r1w = (
            _bd_rowpad(W0[:d]), _bd_rowpad(W0[d : 2 * d]), _bd_rowpad(W0[2 * d :]),
            _tile8(b0), _bd(W1), _tile8(b1),
            _bd_colpad(W2), _tile8(jnp.pad(b2, (0, _P - b2.shape[0]))),
        )
        eA = _edge_mlp3(xiA.reshape(E8h, 128), xjA.reshape(E8h, 128), eaA,
                        *r1w, ea_off=ea_offs[0])
        eB = _edge_mlp3(xiB.reshape(E8h, 128), xjB.reshape(E8h, 128), eaB,
                        *r1w, ea_off=ea_offs[1])
        pA = _sc_scatter_add(eA.reshape(Eh, _P), dstA, zeros)
        pB = _sc_scatter_add(eB.reshape(Eh, _P), dstB, zeros)
        (V0, c0), (V1, c1), (V2, c2) = layer["O"]
        x8 = _node_mlp3(
            x8, pA.reshape(_NC * n8, 128), pB.reshape(_NC * n8, 128),
            _bd_rowpad(V0[:d]), _bd_rowpad(V0[d:]),
            _tile8(c0), _bd(V1), _tile8(c1),
            _bd_colpad(V2), _tile8(jnp.pad(c2, (0, _P - c2.shape[0]))),
        )
        eaA, eaB = eA, eB
        ea_offs = (0, 0)
        d = W2.shape[1]

    (W0, b0), (W1, b1), (W2, b2) = params["R2"]
    x16 = x8.reshape(_NPAD, _P)
    xiA, xjA = _sc_gather2(x16, dstA, srcA)
    xiB, xjB = _sc_gather2(x16, dstB, srcB)
    r2w = (
        _bd_rowpad(W0[:d]), _bd_rowpad(W0[d : 2 * d]), _bd_rowpad(W0[2 * d :]),
        _tile8(b0), _bd(W1), _tile8(b1), _bd(W2), _tile8(b2),
    )
    oA = _edge_mlp3(xiA.reshape(E8h, 128), xjA.reshape(E8h, 128), eaA,
                    *r2w, final=True, ea_off=ea_offs[0], out_rows=E8)
    oB = _edge_mlp3(xiB.reshape(E8h, 128), xjB.reshape(E8h, 128), eaB,
                    *r2w, final=True, ea_off=ea_offs[1],
                    out_rows=E8, out_off=hoff, out_alias=oA)
    return oB.reshape(E, 1)


# speedup vs baseline: 1.2312x; 1.0618x over previous
import functools

import jax
import jax.numpy as jnp
from jax import lax
from jax.experimental import pallas as pl
from jax.experimental.pallas import tpu as pltpu
from jax.experimental.pallas import tpu_sc as plsc

_NC = 2
_NS = 16
_NW = _NC * _NS
_P = 16
_PK = 8
_NPAD = 102400
_BE8 = 2000


def _bd(w):
    return jnp.kron(jnp.eye(_PK, dtype=w.dtype), w)


def _bd_rowpad(w):
    return _bd(jnp.pad(w, ((0, _P - w.shape[0]), (0, 0))))


def _bd_colpad(w):
    return _bd(jnp.pad(w, ((0, 0), (0, _P - w.shape[1]))))


def _tile8(b):
    return jnp.tile(b, _PK)[None]


def _sc_repack(src_t, zeros, n, d_in, out_rows):
    per_w = n // _NW
    C = 2000 if per_w % 2000 == 0 else per_w
    n_chunks = per_w // C
    nb = -(-C // 16)
    tail = out_rows - n
    mesh = plsc.VectorSubcoreMesh(core_axis_name="c", subcore_axis_name="s")

    @functools.partial(
        pl.kernel,
        mesh=mesh,
        out_type=jax.ShapeDtypeStruct((out_rows, _P), jnp.float32),
        scratch_types=[
            pltpu.VMEM((C * d_in,), jnp.float32),
            pltpu.VMEM((C, _P), jnp.float32),
        ],
        compiler_params=pltpu.CompilerParams(
            use_tc_tiling_on_sc=False, needs_layout_passes=False
        ),
    )
    def k(src_h, zeros_h, out_h, bufs, buf16):
        wid = lax.axis_index("s") * _NC + lax.axis_index("c")
        base = wid * per_w
        pltpu.sync_copy(zeros_h.at[pl.ds(0, C)], buf16)
        if tail:
            @pl.when(wid == 0)
            def _():
                pltpu.sync_copy(
                    zeros_h.at[pl.ds(0, tail)], out_h.at[pl.ds(n, tail)]
                )

        def chunk(i, carry):
            row0 = base + i * C
            for kf in range(d_in):
                pltpu.sync_copy(
                    src_h.at[kf, pl.ds(row0, C)],
                    bufs.at[pl.ds(kf * C, C)],
                )

            def batch(kk, c2):
                iota = lax.iota(jnp.int32, 16)
                e = jnp.minimum(kk * 16 + iota, C - 1)
                for j in range(d_in):
                    v = plsc.load_gather(bufs, [j * C + e])
                    plsc.store_scatter(
                        buf16, [e, jnp.full((16,), j, jnp.int32)], v
                    )
                return c2

            lax.fori_loop(0, nb, batch, 0)
            pltpu.sync_copy(buf16, out_h.at[pl.ds(row0, C)])
            return carry

        lax.fori_loop(0, n_chunks, chunk, 0)

    return k(src_t, zeros)


def _sc_gather2(table, idx_a, idx_b):
    E = idx_a.shape[0]
    per_w = E // _NW
    C = 2000 if per_w % 2000 == 0 else 1000
    n_chunks = per_w // C
    mesh = plsc.VectorSubcoreMesh(core_axis_name="c", subcore_axis_name="s")

    @functools.partial(
        pl.kernel,
        mesh=mesh,
        out_type=(
            jax.ShapeDtypeStruct((E, _P), jnp.float32),
            jax.ShapeDtypeStruct((E, _P), jnp.float32),
        ),
        scratch_types=[
            pltpu.VMEM((C,), jnp.int32),
            pltpu.VMEM((C,), jnp.int32),
            pltpu.VMEM((C, _P), jnp.float32),
            pltpu.VMEM((C, _P), jnp.float32),
            pltpu.SemaphoreType.DMA,
            pltpu.SemaphoreType.DMA,
        ],
        compiler_params=pltpu.CompilerParams(use_tc_tiling_on_sc=False),
    )
    def k(table_h, ia_h, ib_h, oa, ob, ia_v, ib_v, ra_v, rb_v, sa, sb):
        table = table_h
        wid = lax.axis_index("s") * _NC + lax.axis_index("c")
        base = pl.multiple_of(wid * per_w, 8)

        def body(i, carry):
            off = pl.multiple_of(base + i * C, 8)
            pltpu.sync_copy(ia_h.at[pl.ds(off, C)], ia_v)
            pltpu.sync_copy(ib_h.at[pl.ds(off, C)], ib_v)
            ca = pltpu.async_copy(table.at[ia_v], ra_v, sa)
            cb = pltpu.async_copy(table.at[ib_v], rb_v, sb)
            ca.wait()
            cb.wait()
            pltpu.sync_copy(ra_v, oa.at[pl.ds(off, C)])
            pltpu.sync_copy(rb_v, ob.at[pl.ds(off, C)])
            return carry

        lax.fori_loop(0, n_chunks, body, 0)

    return k(table, idx_a, idx_b)


def _sc_scatter_add(vals, idx, zeros):
    E = idx.shape[0]
    per_core = E // _NC
    per_sub = per_core // _NS
    C = 1000
    n_chunks = per_sub // C
    rows = _NPAD // _NS
    mesh = plsc.VectorSubcoreMesh(core_axis_name="c", subcore_axis_name="s")

    @functools.partial(
        pl.kernel,
        mesh=mesh,
        out_type=jax.ShapeDtypeStruct((_NC * _NPAD, _P), jnp.float32),
        scratch_types=[
            pltpu.VMEM((C,), jnp.int32),
            pltpu.VMEM((C, _P), jnp.float32),
            pltpu.VMEM_SHARED((_NPAD, _P), jnp.float32),
            pltpu.SemaphoreType.DMA,
        ],
        compiler_params=pltpu.CompilerParams(use_tc_tiling_on_sc=False),
    )
    def k(vals_h, idx_h, zeros_h, out_h, idx_v, vals_v, acc, sem):
        vals = vals_h
        zeros = zeros_h
        out = out_h
        cid = lax.axis_index("c")
        sid = lax.axis_index("s")
        pltpu.sync_copy(
            zeros.at[pl.ds(sid * rows, rows)], acc.at[pl.ds(sid * rows, rows)]
        )
        plsc.subcore_barrier()
        base = cid * per_core + sid * per_sub

        def body(i, carry):
            off = pl.multiple_of(base + i * C, 8)
            pltpu.sync_copy(idx_h.at[pl.ds(off, C)], idx_v)
            pltpu.sync_copy(vals.at[pl.ds(off, C)], vals_v)
            pltpu.sync_copy(vals_v, acc.at[idx_v], add=True)
            return carry

        lax.fori_loop(0, n_chunks, body, 0)
        plsc.subcore_barrier()
        pltpu.sync_copy(
            acc.at[pl.ds(sid * rows, rows)],
            out.at[pl.ds(cid * _NPAD + sid * rows, rows)],
        )

    return k(vals, idx, zeros)


def _edge_mlp3(xi8, xj8, ea8, Wi, Wj, We, b0, W1, b1, W2, b2, final=False,
               ea_off=0, out_off=0, out_rows=None, out_alias=None):
    E8 = xi8.shape[0]
    H8 = W1.shape[0]
    do8 = W2.shape[1]
    BE8 = _BE8
    grid = E8 // BE8
    if out_rows is None:
        out_rows = E8

    def body(xi_r, xj_r, ea_r, wi, wj, we, b0r, w1, b1r, w2, b2r, *rest):
        o_r = rest[-1]
        h = (
            jnp.dot(xi_r[...], wi[...], preferred_element_type=jnp.float32)
            + jnp.dot(xj_r[...], wj[...], preferred_element_type=jnp.float32)
            + jnp.dot(ea_r[...], we[...], preferred_element_type=jnp.float32)
            + b0r[...]
        )
        h = jnp.maximum(h, 0.0)
        h = jnp.maximum(
            jnp.dot(h, w1[...], preferred_element_type=jnp.float32) + b1r[...], 0.0
        )
        o = jnp.dot(h, w2[...], preferred_element_type=jnp.float32) + b2r[...]
        if final:
            o = jax.nn.sigmoid(o)
        o_r[...] = o

    wspec = lambda s: pl.BlockSpec(s, lambda i: (0, 0))
    do = _PK if final else 128
    out_shape = jax.ShapeDtypeStruct((out_rows, do), jnp.float32)
    out_spec = pl.BlockSpec((BE8, do), lambda i: (out_off + i, 0))
    args = [xi8, xj8, ea8, Wi, Wj, We, b0, W1, b1, W2, b2]
    in_specs = [
        pl.BlockSpec((BE8, 128), lambda i: (i, 0)),
        pl.BlockSpec((BE8, 128), lambda i: (i, 0)),
        pl.BlockSpec((BE8, 128), lambda i: (ea_off + i, 0)),
        wspec((128, H8)),
        wspec((128, H8)),
        wspec((128, H8)),
        wspec((1, H8)),
        wspec((H8, H8)),
        wspec((1, H8)),
        wspec((H8, do8)),
        wspec((1, do8)),
    ]
    aliases = {}
    if out_alias is not None:
        args.append(out_alias)
        in_specs.append(pl.BlockSpec((BE8, do), lambda i: (out_off + i, 0)))
        aliases = {len(args) - 1: 0}
    return pl.pallas_call(
        body,
        grid=(grid,),
        in_specs=in_specs,
        out_specs=out_spec,
        out_shape=out_shape,
        input_output_aliases=aliases,
    )(*args)


def _node_mlp3(x8, pa8, pb8, Wx, Wa, b0, W1, b1, W2, b2):
    N8 = x8.shape[0]
    H8 = W1.shape[0]
    BN8 = 800
    grid = N8 // BN8

    def body(x_r, p0_r, p1_r, p2_r, p3_r, wx, wa, b0r, w1, b1r, w2, b2r, o_r):
        aggr = p0_r[...] + p1_r[...] + p2_r[...] + p3_r[...]
        h = (
            jnp.dot(x_r[...], wx[...], preferred_element_type=jnp.float32)
            + jnp.dot(aggr, wa[...], preferred_element_type=jnp.float32)
            + b0r[...]
        )
        h = jnp.maximum(h, 0.0)
        h = jnp.maximum(
            jnp.dot(h, w1[...], preferred_element_type=jnp.float32) + b1r[...], 0.0
        )
        o_r[...] = jnp.dot(h, w2[...], preferred_element_type=jnp.float32) + b2r[...]

    wspec = lambda s: pl.BlockSpec(s, lambda i: (0, 0))
    nblocks = N8 // BN8
    return pl.pallas_call(
        body,
        grid=(grid,),
        in_specs=[
            pl.BlockSpec((BN8, 128), lambda i: (i, 0)),
            pl.BlockSpec((BN8, 128), lambda i: (i, 0)),
            pl.BlockSpec((BN8, 128), lambda i: (nblocks + i, 0)),
            pl.BlockSpec((BN8, 128), lambda i: (i, 0)),
            pl.BlockSpec((BN8, 128), lambda i: (nblocks + i, 0)),
            wspec((128, H8)),
            wspec((128, H8)),
            wspec((1, H8)),
            wspec((H8, H8)),
            wspec((1, H8)),
            wspec((H8, 128)),
            wspec((1, 128)),
        ],
        out_specs=pl.BlockSpec((BN8, 128), lambda i: (i, 0)),
        out_shape=jax.ShapeDtypeStruct((N8, 128), jnp.float32),
    )(x8, pa8, pa8, pb8, pb8, Wx, Wa, b0, W1, b1, W2, b2)


def kernel(x, edge_index, edge_attr, params):
    src = edge_index[0].astype(jnp.int32)
    dst = edge_index[1].astype(jnp.int32)
    N = x.shape[0]
    E = edge_attr.shape[0]
    E8 = E // _PK
    n8 = _NPAD // _PK

    zeros = jnp.zeros((_NPAD, _P), jnp.float32)
    n_work = -(-N // (8 * _NW)) * 8 * _NW
    xt = jnp.pad(x.T, ((0, 0), (0, n_work - N)))
    x8 = _sc_repack(xt, zeros, n_work, 3, _NPAD).reshape(n8, 128)
    ea8 = _sc_repack(edge_attr.T, zeros, E, 3, E).reshape(E8, 128)
    d = 3

    Eh = E // 2
    E8h = E8 // 2
    hoff = E8h // _BE8
    srcA, srcB = src[:Eh], src[Eh:]
    dstA, dstB = dst[:Eh], dst[Eh:]
    eaA, eaB = ea8, ea8
    ea_offs = (0, hoff)

    for name in ("IN1", "IN2", "IN3"):
        layer = params[name]
        (W0, b0), (W1, b1), (W2, b2) = layer["R1"]
        x16 = x8.reshape(_NPAD, _P)
        xiA, xjA = _sc_gather2(x16, dstA, srcA)
        xiB, xjB = _sc_gather2(x16, dstB, srcB)
        r1w = (
            _bd_rowpad(W0[:d]), _bd_rowpad(W0[d : 2 * d]), _bd_rowpad(W0[2 * d :]),
            _tile8(b0), _bd(W1), _tile8(b1),
            _bd_colpad(W2), _tile8(jnp.pad(b2, (0, _P - b2.shape[0]))),
        )
        eA = _edge_mlp3(xiA.reshape(E8h, 128), xjA.reshape(E8h, 128), eaA,
                        *r1w, ea_off=ea_offs[0])
        eB = _edge_mlp3(xiB.reshape(E8h, 128), xjB.reshape(E8h, 128), eaB,
                        *r1w, ea_off=ea_offs[1])
        pA = _sc_scatter_add(eA.reshape(Eh, _P), dstA, zeros)
        pB = _sc_scatter_add(eB.reshape(Eh, _P), dstB, zeros)
        (V0, c0), (V1, c1), (V2, c2) = layer["O"]
        x8 = _node_mlp3(
            x8, pA.reshape(_NC * n8, 128), pB.reshape(_NC * n8, 128),
            _bd_rowpad(V0[:d]), _bd_rowpad(V0[d:]),
            _tile8(c0), _bd(V1), _tile8(c1),
            _bd_colpad(V2), _tile8(jnp.pad(c2, (0, _P - c2.shape[0]))),
        )
        eaA, eaB = eA, eB
        ea_offs = (0, 0)
        d = W2.shape[1]

    (W0, b0), (W1, b1), (W2, b2) = params["R2"]
    x16 = x8.reshape(_NPAD, _P)
    xiA, xjA = _sc_gather2(x16, dstA, srcA)
    xiB, xjB = _sc_gather2(x16, dstB, srcB)
    r2w = (
        _bd_rowpad(W0[:d]), _bd_rowpad(W0[d : 2 * d]), _bd_rowpad(W0[2 * d :]),
        _tile8(b0), _bd(W1), _tile8(b1), _bd(W2), _tile8(b2),
    )
    oA = _edge_mlp3(xiA.reshape(E8h, 128), xjA.reshape(E8h, 128), eaA,
                    *r2w, final=True, ea_off=ea_offs[0], out_rows=E8)
    oB = _edge_mlp3(xiB.reshape(E8h, 128), xjB.reshape(E8h, 128), eaB,
                    *r2w, final=True, ea_off=ea_offs[1],
                    out_rows=E8, out_off=hoff, out_alias=oA)
    return oB.reshape(E, 1)


# speedup vs baseline: 1.2550x; 1.0193x over previous
import functools

import jax
import jax.numpy as jnp
from jax import lax
from jax.experimental import pallas as pl
from jax.experimental.pallas import tpu as pltpu
from jax.experimental.pallas import tpu_sc as plsc

_NC = 2
_NS = 16
_NW = _NC * _NS
_P = 16
_PK = 8
_NPAD = 102400
_BE8 = 4000


def _bd(w):
    return jnp.kron(jnp.eye(_PK, dtype=w.dtype), w)


def _bd_rowpad(w):
    return _bd(jnp.pad(w, ((0, _P - w.shape[0]), (0, 0))))


def _bd_colpad(w):
    return _bd(jnp.pad(w, ((0, 0), (0, _P - w.shape[1]))))


def _tile8(b):
    return jnp.tile(b, _PK)[None]


def _sc_repack(src_t, zeros, n, d_in, out_rows):
    per_w = n // _NW
    C = 2000 if per_w % 2000 == 0 else per_w
    n_chunks = per_w // C
    nb = -(-C // 16)
    tail = out_rows - n
    mesh = plsc.VectorSubcoreMesh(core_axis_name="c", subcore_axis_name="s")

    @functools.partial(
        pl.kernel,
        mesh=mesh,
        out_type=jax.ShapeDtypeStruct((out_rows, _P), jnp.float32),
        scratch_types=[
            pltpu.VMEM((C * d_in,), jnp.float32),
            pltpu.VMEM((C, _P), jnp.float32),
        ],
        compiler_params=pltpu.CompilerParams(
            use_tc_tiling_on_sc=False, needs_layout_passes=False
        ),
    )
    def k(src_h, zeros_h, out_h, bufs, buf16):
        wid = lax.axis_index("s") * _NC + lax.axis_index("c")
        base = wid * per_w
        pltpu.sync_copy(zeros_h.at[pl.ds(0, C)], buf16)
        if tail:
            @pl.when(wid == 0)
            def _():
                pltpu.sync_copy(
                    zeros_h.at[pl.ds(0, tail)], out_h.at[pl.ds(n, tail)]
                )

        def chunk(i, carry):
            row0 = base + i * C
            for kf in range(d_in):
                pltpu.sync_copy(
                    src_h.at[kf, pl.ds(row0, C)],
                    bufs.at[pl.ds(kf * C, C)],
                )

            def batch(kk, c2):
                iota = lax.iota(jnp.int32, 16)
                e = jnp.minimum(kk * 16 + iota, C - 1)
                for j in range(d_in):
                    v = plsc.load_gather(bufs, [j * C + e])
                    plsc.store_scatter(
                        buf16, [e, jnp.full((16,), j, jnp.int32)], v
                    )
                return c2

            lax.fori_loop(0, nb, batch, 0)
            pltpu.sync_copy(buf16, out_h.at[pl.ds(row0, C)])
            return carry

        lax.fori_loop(0, n_chunks, chunk, 0)

    return k(src_t, zeros)


def _sc_gather2(table, idx_a, idx_b):
    E = idx_a.shape[0]
    per_w = E // _NW
    C = 2000 if per_w % 2000 == 0 else 1000
    n_chunks = per_w // C
    mesh = plsc.VectorSubcoreMesh(core_axis_name="c", subcore_axis_name="s")

    @functools.partial(
        pl.kernel,
        mesh=mesh,
        out_type=(
            jax.ShapeDtypeStruct((E, _P), jnp.float32),
            jax.ShapeDtypeStruct((E, _P), jnp.float32),
        ),
        scratch_types=[
            pltpu.VMEM((C,), jnp.int32),
            pltpu.VMEM((C,), jnp.int32),
            pltpu.VMEM((C, _P), jnp.float32),
            pltpu.VMEM((C, _P), jnp.float32),
            pltpu.SemaphoreType.DMA,
            pltpu.SemaphoreType.DMA,
        ],
        compiler_params=pltpu.CompilerParams(use_tc_tiling_on_sc=False),
    )
    def k(table_h, ia_h, ib_h, oa, ob, ia_v, ib_v, ra_v, rb_v, sa, sb):
        table = table_h
        wid = lax.axis_index("s") * _NC + lax.axis_index("c")
        base = pl.multiple_of(wid * per_w, 8)

        def body(i, carry):
            off = pl.multiple_of(base + i * C, 8)
            pltpu.sync_copy(ia_h.at[pl.ds(off, C)], ia_v)
            pltpu.sync_copy(ib_h.at[pl.ds(off, C)], ib_v)
            ca = pltpu.async_copy(table.at[ia_v], ra_v, sa)
            cb = pltpu.async_copy(table.at[ib_v], rb_v, sb)
            ca.wait()
            cb.wait()
            pltpu.sync_copy(ra_v, oa.at[pl.ds(off, C)])
            pltpu.sync_copy(rb_v, ob.at[pl.ds(off, C)])
            return carry

        lax.fori_loop(0, n_chunks, body, 0)

    return k(table, idx_a, idx_b)


def _sc_scatter_add(vals, idx, zeros):
    E = idx.shape[0]
    per_core = E // _NC
    per_sub = per_core // _NS
    C = 1000
    n_chunks = per_sub // C
    rows = _NPAD // _NS
    mesh = plsc.VectorSubcoreMesh(core_axis_name="c", subcore_axis_name="s")

    @functools.partial(
        pl.kernel,
        mesh=mesh,
        out_type=jax.ShapeDtypeStruct((_NC * _NPAD, _P), jnp.float32),
        scratch_types=[
            pltpu.VMEM((C,), jnp.int32),
            pltpu.VMEM((C, _P), jnp.float32),
            pltpu.VMEM_SHARED((_NPAD, _P), jnp.float32),
            pltpu.SemaphoreType.DMA,
        ],
        compiler_params=pltpu.CompilerParams(use_tc_tiling_on_sc=False),
    )
    def k(vals_h, idx_h, zeros_h, out_h, idx_v, vals_v, acc, sem):
        vals = vals_h
        zeros = zeros_h
        out = out_h
        cid = lax.axis_index("c")
        sid = lax.axis_index("s")
        pltpu.sync_copy(
            zeros.at[pl.ds(sid * rows, rows)], acc.at[pl.ds(sid * rows, rows)]
        )
        plsc.subcore_barrier()
        base = cid * per_core + sid * per_sub

        def body(i, carry):
            off = pl.multiple_of(base + i * C, 8)
            pltpu.sync_copy(idx_h.at[pl.ds(off, C)], idx_v)
            pltpu.sync_copy(vals.at[pl.ds(off, C)], vals_v)
            pltpu.sync_copy(vals_v, acc.at[idx_v], add=True)
            return carry

        lax.fori_loop(0, n_chunks, body, 0)
        plsc.subcore_barrier()
        pltpu.sync_copy(
            acc.at[pl.ds(sid * rows, rows)],
            out.at[pl.ds(cid * _NPAD + sid * rows, rows)],
        )

    return k(vals, idx, zeros)


def _edge_mlp3(xi8, xj8, ea8, Wi, Wj, We, b0, W1, b1, W2, b2, final=False,
               ea_off=0, out_off=0, out_rows=None, out_alias=None):
    E8 = xi8.shape[0]
    H8 = W1.shape[0]
    do8 = W2.shape[1]
    BE8 = _BE8
    grid = E8 // BE8
    if out_rows is None:
        out_rows = E8

    def body(xi_r, xj_r, ea_r, wi, wj, we, b0r, w1, b1r, w2, b2r, *rest):
        o_r = rest[-1]
        h = (
            jnp.dot(xi_r[...], wi[...], preferred_element_type=jnp.float32)
            + jnp.dot(xj_r[...], wj[...], preferred_element_type=jnp.float32)
            + jnp.dot(ea_r[...], we[...], preferred_element_type=jnp.float32)
            + b0r[...]
        )
        h = jnp.maximum(h, 0.0)
        h = jnp.maximum(
            jnp.dot(h, w1[...], preferred_element_type=jnp.float32) + b1r[...], 0.0
        )
        o = jnp.dot(h, w2[...], preferred_element_type=jnp.float32) + b2r[...]
        if final:
            o = jax.nn.sigmoid(o)
        o_r[...] = o

    wspec = lambda s: pl.BlockSpec(s, lambda i: (0, 0))
    do = _PK if final else 128
    out_shape = jax.ShapeDtypeStruct((out_rows, do), jnp.float32)
    out_spec = pl.BlockSpec((BE8, do), lambda i: (out_off + i, 0))
    args = [xi8, xj8, ea8, Wi, Wj, We, b0, W1, b1, W2, b2]
    in_specs = [
        pl.BlockSpec((BE8, 128), lambda i: (i, 0)),
        pl.BlockSpec((BE8, 128), lambda i: (i, 0)),
        pl.BlockSpec((BE8, 128), lambda i: (ea_off + i, 0)),
        wspec((128, H8)),
        wspec((128, H8)),
        wspec((128, H8)),
        wspec((1, H8)),
        wspec((H8, H8)),
        wspec((1, H8)),
        wspec((H8, do8)),
        wspec((1, do8)),
    ]
    aliases = {}
    if out_alias is not None:
        args.append(out_alias)
        in_specs.append(pl.BlockSpec((BE8, do), lambda i: (out_off + i, 0)))
        aliases = {len(args) - 1: 0}
    return pl.pallas_call(
        body,
        grid=(grid,),
        in_specs=in_specs,
        out_specs=out_spec,
        out_shape=out_shape,
        input_output_aliases=aliases,
    )(*args)


def _node_mlp3(x8, pa8, pb8, Wx, Wa, b0, W1, b1, W2, b2):
    N8 = x8.shape[0]
    H8 = W1.shape[0]
    BN8 = 800
    grid = N8 // BN8

    def body(x_r, p0_r, p1_r, p2_r, p3_r, wx, wa, b0r, w1, b1r, w2, b2r, o_r):
        aggr = p0_r[...] + p1_r[...] + p2_r[...] + p3_r[...]
        h = (
            jnp.dot(x_r[...], wx[...], preferred_element_type=jnp.float32)
            + jnp.dot(aggr, wa[...], preferred_element_type=jnp.float32)
            + b0r[...]
        )
        h = jnp.maximum(h, 0.0)
        h = jnp.maximum(
            jnp.dot(h, w1[...], preferred_element_type=jnp.float32) + b1r[...], 0.0
        )
        o_r[...] = jnp.dot(h, w2[...], preferred_element_type=jnp.float32) + b2r[...]

    wspec = lambda s: pl.BlockSpec(s, lambda i: (0, 0))
    nblocks = N8 // BN8
    return pl.pallas_call(
        body,
        grid=(grid,),
        in_specs=[
            pl.BlockSpec((BN8, 128), lambda i: (i, 0)),
            pl.BlockSpec((BN8, 128), lambda i: (i, 0)),
            pl.BlockSpec((BN8, 128), lambda i: (nblocks + i, 0)),
            pl.BlockSpec((BN8, 128), lambda i: (i, 0)),
            pl.BlockSpec((BN8, 128), lambda i: (nblocks + i, 0)),
            wspec((128, H8)),
            wspec((128, H8)),
            wspec((1, H8)),
            wspec((H8, H8)),
            wspec((1, H8)),
            wspec((H8, 128)),
            wspec((1, 128)),
        ],
        out_specs=pl.BlockSpec((BN8, 128), lambda i: (i, 0)),
        out_shape=jax.ShapeDtypeStruct((N8, 128), jnp.float32),
    )(x8, pa8, pa8, pb8, pb8, Wx, Wa, b0, W1, b1, W2, b2)


def kernel(x, edge_index, edge_attr, params):
    src = edge_index[0].astype(jnp.int32)
    dst = edge_index[1].astype(jnp.int32)
    N = x.shape[0]
    E = edge_attr.shape[0]
    E8 = E // _PK
    n8 = _NPAD // _PK

    zeros = jnp.zeros((_NPAD, _P), jnp.float32)
    n_work = -(-N // (8 * _NW)) * 8 * _NW
    xt = jnp.pad(x.T, ((0, 0), (0, n_work - N)))
    x8 = _sc_repack(xt, zeros, n_work, 3, _NPAD).reshape(n8, 128)
    ea8 = _sc_repack(edge_attr.T, zeros, E, 3, E).reshape(E8, 128)
    d = 3

    Eh = E // 2
    E8h = E8 // 2
    hoff = E8h // _BE8
    srcA, srcB = src[:Eh], src[Eh:]
    dstA, dstB = dst[:Eh], dst[Eh:]
    eaA, eaB = ea8, ea8
    ea_offs = (0, hoff)

    for name in ("IN1", "IN2", "IN3"):
        layer = params[name]
        (W0, b0), (W1, b1), (W2, b2) = layer["R1"]
        x16 = x8.reshape(_NPAD, _P)
        xiA, xjA = _sc_gather2(x16, dstA, srcA)
        xiB, xjB = _sc_gather2(x16, dstB, srcB)
        r1w = (
            _bd_rowpad(W0[:d]), _bd_rowpad(W0[d : 2 * d]), _bd_rowpad(W0[2 * d :]),
            _tile8(b0), _bd(W1), _tile8(b1),
            _bd_colpad(W2), _tile8(jnp.pad(b2, (0, _P - b2.shape[0]))),
        )
        eA = _edge_mlp3(xiA.reshape(E8h, 128), xjA.reshape(E8h, 128), eaA,
                        *r1w, ea_off=ea_offs[0])
        eB = _edge_mlp3(xiB.reshape(E8h, 128), xjB.reshape(E8h, 128), eaB,
                        *r1w, ea_off=ea_offs[1])
        pA = _sc_scatter_add(eA.reshape(Eh, _P), dstA, zeros)
        pB = _sc_scatter_add(eB.reshape(Eh, _P), dstB, zeros)
        (V0, c0), (V1, c1), (V2, c2) = layer["O"]
        x8 = _node_mlp3(
            x8, pA.reshape(_NC * n8, 128), pB.reshape(_NC * n8, 128),
            _bd_rowpad(V0[:d]), _bd_rowpad(V0[d:]),
            _tile8(c0), _bd(V1), _tile8(c1),
            _bd_colpad(V2), _tile8(jnp.pad(c2, (0, _P - c2.shape[0]))),
        )
        eaA, eaB = eA, eB
        ea_offs = (0, 0)
        d = W2.shape[1]

    (W0, b0), (W1, b1), (W2, b2) = params["R2"]
    x16 = x8.reshape(_NPAD, _P)
    xiA, xjA = _sc_gather2(x16, dstA, srcA)
    xiB, xjB = _sc_gather2(x16, dstB, srcB)
    r2w = (
        _bd_rowpad(W0[:d]), _bd_rowpad(W0[d : 2 * d]), _bd_rowpad(W0[2 * d :]),
        _tile8(b0), _bd(W1), _tile8(b1), _bd(W2), _tile8(b2),
    )
    oA = _edge_mlp3(xiA.reshape(E8h, 128), xjA.reshape(E8h, 128), eaA,
                    *r2w, final=True, ea_off=ea_offs[0], out_rows=E8)
    oB = _edge_mlp3(xiB.reshape(E8h, 128), xjB.reshape(E8h, 128), eaB,
                    *r2w, final=True, ea_off=ea_offs[1],
                    out_rows=E8, out_off=hoff, out_alias=oA)
    return oB.reshape(E, 1)


# speedup vs baseline: 1.2589x; 1.0031x over previous
import functools

import jax
import jax.numpy as jnp
from jax import lax
from jax.experimental import pallas as pl
from jax.experimental.pallas import tpu as pltpu
from jax.experimental.pallas import tpu_sc as plsc

_NC = 2
_NS = 16
_NW = _NC * _NS
_P = 16
_PK = 8
_NPAD = 102400
_BE8 = 5000


def _bd(w):
    return jnp.kron(jnp.eye(_PK, dtype=w.dtype), w)


def _bd_rowpad(w):
    return _bd(jnp.pad(w, ((0, _P - w.shape[0]), (0, 0))))


def _bd_colpad(w):
    return _bd(jnp.pad(w, ((0, 0), (0, _P - w.shape[1]))))


def _tile8(b):
    return jnp.tile(b, _PK)[None]


def _sc_repack(src_t, zeros, n, d_in, out_rows):
    per_w = n // _NW
    C = 2000 if per_w % 2000 == 0 else per_w
    n_chunks = per_w // C
    nb = -(-C // 16)
    tail = out_rows - n
    mesh = plsc.VectorSubcoreMesh(core_axis_name="c", subcore_axis_name="s")

    @functools.partial(
        pl.kernel,
        mesh=mesh,
        out_type=jax.ShapeDtypeStruct((out_rows, _P), jnp.float32),
        scratch_types=[
            pltpu.VMEM((C * d_in,), jnp.float32),
            pltpu.VMEM((C, _P), jnp.float32),
        ],
        compiler_params=pltpu.CompilerParams(
            use_tc_tiling_on_sc=False, needs_layout_passes=False
        ),
    )
    def k(src_h, zeros_h, out_h, bufs, buf16):
        wid = lax.axis_index("s") * _NC + lax.axis_index("c")
        base = wid * per_w
        pltpu.sync_copy(zeros_h.at[pl.ds(0, C)], buf16)
        if tail:
            @pl.when(wid == 0)
            def _():
                pltpu.sync_copy(
                    zeros_h.at[pl.ds(0, tail)], out_h.at[pl.ds(n, tail)]
                )

        def chunk(i, carry):
            row0 = base + i * C
            for kf in range(d_in):
                pltpu.sync_copy(
                    src_h.at[kf, pl.ds(row0, C)],
                    bufs.at[pl.ds(kf * C, C)],
                )

            def batch(kk, c2):
                iota = lax.iota(jnp.int32, 16)
                e = jnp.minimum(kk * 16 + iota, C - 1)
                for j in range(d_in):
                    v = plsc.load_gather(bufs, [j * C + e])
                    plsc.store_scatter(
                        buf16, [e, jnp.full((16,), j, jnp.int32)], v
                    )
                return c2

            lax.fori_loop(0, nb, batch, 0)
            pltpu.sync_copy(buf16, out_h.at[pl.ds(row0, C)])
            return carry

        lax.fori_loop(0, n_chunks, chunk, 0)

    return k(src_t, zeros)


def _sc_gather2(table, idx_a, idx_b):
    E = idx_a.shape[0]
    per_w = E // _NW
    C = 2000 if per_w % 2000 == 0 else 1000
    n_chunks = per_w // C
    mesh = plsc.VectorSubcoreMesh(core_axis_name="c", subcore_axis_name="s")

    @functools.partial(
        pl.kernel,
        mesh=mesh,
        out_type=(
            jax.ShapeDtypeStruct((E, _P), jnp.float32),
            jax.ShapeDtypeStruct((E, _P), jnp.float32),
        ),
        scratch_types=[
            pltpu.VMEM((C,), jnp.int32),
            pltpu.VMEM((C,), jnp.int32),
            pltpu.VMEM((C, _P), jnp.float32),
            pltpu.VMEM((C, _P), jnp.float32),
            pltpu.SemaphoreType.DMA,
            pltpu.SemaphoreType.DMA,
        ],
        compiler_params=pltpu.CompilerParams(use_tc_tiling_on_sc=False),
    )
    def k(table_h, ia_h, ib_h, oa, ob, ia_v, ib_v, ra_v, rb_v, sa, sb):
        table = table_h
        wid = lax.axis_index("s") * _NC + lax.axis_index("c")
        base = pl.multiple_of(wid * per_w, 8)

        def body(i, carry):
            off = pl.multiple_of(base + i * C, 8)
            pltpu.sync_copy(ia_h.at[pl.ds(off, C)], ia_v)
            pltpu.sync_copy(ib_h.at[pl.ds(off, C)], ib_v)
            ca = pltpu.async_copy(table.at[ia_v], ra_v, sa)
            cb = pltpu.async_copy(table.at[ib_v], rb_v, sb)
            ca.wait()
            cb.wait()
            pltpu.sync_copy(ra_v, oa.at[pl.ds(off, C)])
            pltpu.sync_copy(rb_v, ob.at[pl.ds(off, C)])
            return carry

        lax.fori_loop(0, n_chunks, body, 0)

    return k(table, idx_a, idx_b)


def _sc_scatter_add(vals, idx, zeros):
    E = idx.shape[0]
    per_core = E // _NC
    per_sub = per_core // _NS
    C = 1000
    n_chunks = per_sub // C
    rows = _NPAD // _NS
    mesh = plsc.VectorSubcoreMesh(core_axis_name="c", subcore_axis_name="s")

    @functools.partial(
        pl.kernel,
        mesh=mesh,
        out_type=jax.ShapeDtypeStruct((_NC * _NPAD, _P), jnp.float32),
        scratch_types=[
            pltpu.VMEM((C,), jnp.int32),
            pltpu.VMEM((C, _P), jnp.float32),
            pltpu.VMEM_SHARED((_NPAD, _P), jnp.float32),
            pltpu.SemaphoreType.DMA,
        ],
        compiler_params=pltpu.CompilerParams(use_tc_tiling_on_sc=False),
    )
    def k(vals_h, idx_h, zeros_h, out_h, idx_v, vals_v, acc, sem):
        vals = vals_h
        zeros = zeros_h
        out = out_h
        cid = lax.axis_index("c")
        sid = lax.axis_index("s")
        pltpu.sync_copy(
            zeros.at[pl.ds(sid * rows, rows)], acc.at[pl.ds(sid * rows, rows)]
        )
        plsc.subcore_barrier()
        base = cid * per_core + sid * per_sub

        def body(i, carry):
            off = pl.multiple_of(base + i * C, 8)
            pltpu.sync_copy(idx_h.at[pl.ds(off, C)], idx_v)
            pltpu.sync_copy(vals.at[pl.ds(off, C)], vals_v)
            pltpu.sync_copy(vals_v, acc.at[idx_v], add=True)
            return carry

        lax.fori_loop(0, n_chunks, body, 0)
        plsc.subcore_barrier()
        pltpu.sync_copy(
            acc.at[pl.ds(sid * rows, rows)],
            out.at[pl.ds(cid * _NPAD + sid * rows, rows)],
        )

    return k(vals, idx, zeros)


def _edge_mlp3(xi8, xj8, ea8, Wi, Wj, We, b0, W1, b1, W2, b2, final=False,
               ea_off=0, out_off=0, out_rows=None, out_alias=None):
    E8 = xi8.shape[0]
    H8 = W1.shape[0]
    do8 = W2.shape[1]
    BE8 = _BE8
    grid = E8 // BE8
    if out_rows is None:
        out_rows = E8

    def body(xi_r, xj_r, ea_r, wi, wj, we, b0r, w1, b1r, w2, b2r, *rest):
        o_r = rest[-1]
        h = (
            jnp.dot(xi_r[...], wi[...], preferred_element_type=jnp.float32)
            + jnp.dot(xj_r[...], wj[...], preferred_element_type=jnp.float32)
            + jnp.dot(ea_r[...], we[...], preferred_element_type=jnp.float32)
            + b0r[...]
        )
        h = jnp.maximum(h, 0.0)
        h = jnp.maximum(
            jnp.dot(h, w1[...], preferred_element_type=jnp.float32) + b1r[...], 0.0
        )
        o = jnp.dot(h, w2[...], preferred_element_type=jnp.float32) + b2r[...]
        if final:
            o = jax.nn.sigmoid(o)
        o_r[...] = o

    wspec = lambda s: pl.BlockSpec(s, lambda i: (0, 0))
    do = _PK if final else 128
    out_shape = jax.ShapeDtypeStruct((out_rows, do), jnp.float32)
    out_spec = pl.BlockSpec((BE8, do), lambda i: (out_off + i, 0))
    args = [xi8, xj8, ea8, Wi, Wj, We, b0, W1, b1, W2, b2]
    in_specs = [
        pl.BlockSpec((BE8, 128), lambda i: (i, 0)),
        pl.BlockSpec((BE8, 128), lambda i: (i, 0)),
        pl.BlockSpec((BE8, 128), lambda i: (ea_off + i, 0)),
        wspec((128, H8)),
        wspec((128, H8)),
        wspec((128, H8)),
        wspec((1, H8)),
        wspec((H8, H8)),
        wspec((1, H8)),
        wspec((H8, do8)),
        wspec((1, do8)),
    ]
    aliases = {}
    if out_alias is not None:
        args.append(out_alias)
        in_specs.append(pl.BlockSpec((BE8, do), lambda i: (out_off + i, 0)))
        aliases = {len(args) - 1: 0}
    return pl.pallas_call(
        body,
        grid=(grid,),
        in_specs=in_specs,
        out_specs=out_spec,
        out_shape=out_shape,
        input_output_aliases=aliases,
    )(*args)


def _node_mlp3(x8, pa8, pb8, Wx, Wa, b0, W1, b1, W2, b2):
    N8 = x8.shape[0]
    H8 = W1.shape[0]
    BN8 = 800
    grid = N8 // BN8

    def body(x_r, p0_r, p1_r, p2_r, p3_r, wx, wa, b0r, w1, b1r, w2, b2r, o_r):
        aggr = p0_r[...] + p1_r[...] + p2_r[...] + p3_r[...]
        h = (
            jnp.dot(x_r[...], wx[...], preferred_element_type=jnp.float32)
            + jnp.dot(aggr, wa[...], preferred_element_type=jnp.float32)
            + b0r[...]
        )
        h = jnp.maximum(h, 0.0)
        h = jnp.maximum(
            jnp.dot(h, w1[...], preferred_element_type=jnp.float32) + b1r[...], 0.0
        )
        o_r[...] = jnp.dot(h, w2[...], preferred_element_type=jnp.float32) + b2r[...]

    wspec = lambda s: pl.BlockSpec(s, lambda i: (0, 0))
    nblocks = N8 // BN8
    return pl.pallas_call(
        body,
        grid=(grid,),
        in_specs=[
            pl.BlockSpec((BN8, 128), lambda i: (i, 0)),
            pl.BlockSpec((BN8, 128), lambda i: (i, 0)),
            pl.BlockSpec((BN8, 128), lambda i: (nblocks + i, 0)),
            pl.BlockSpec((BN8, 128), lambda i: (i, 0)),
            pl.BlockSpec((BN8, 128), lambda i: (nblocks + i, 0)),
            wspec((128, H8)),
            wspec((128, H8)),
            wspec((1, H8)),
            wspec((H8, H8)),
            wspec((1, H8)),
            wspec((H8, 128)),
            wspec((1, 128)),
        ],
        out_specs=pl.BlockSpec((BN8, 128), lambda i: (i, 0)),
        out_shape=jax.ShapeDtypeStruct((N8, 128), jnp.float32),
    )(x8, pa8, pa8, pb8, pb8, Wx, Wa, b0, W1, b1, W2, b2)


def kernel(x, edge_index, edge_attr, params):
    src = edge_index[0].astype(jnp.int32)
    dst = edge_index[1].astype(jnp.int32)
    N = x.shape[0]
    E = edge_attr.shape[0]
    E8 = E // _PK
    n8 = _NPAD // _PK

    zeros = jnp.zeros((_NPAD, _P), jnp.float32)
    n_work = -(-N // (8 * _NW)) * 8 * _NW
    xt = jnp.pad(x.T, ((0, 0), (0, n_work - N)))
    x8 = _sc_repack(xt, zeros, n_work, 3, _NPAD).reshape(n8, 128)
    ea8 = _sc_repack(edge_attr.T, zeros, E, 3, E).reshape(E8, 128)
    d = 3

    Eh = E // 2
    E8h = E8 // 2
    hoff = E8h // _BE8
    srcA, srcB = src[:Eh], src[Eh:]
    dstA, dstB = dst[:Eh], dst[Eh:]
    eaA, eaB = ea8, ea8
    ea_offs = (0, hoff)

    for name in ("IN1", "IN2", "IN3"):
        layer = params[name]
        (W0, b0), (W1, b1), (W2, b2) = layer["R1"]
        x16 = x8.reshape(_NPAD, _P)
        xiA, xjA = _sc_gather2(x16, dstA, srcA)
        xiB, xjB = _sc_gather2(x16, dstB, srcB)
        r1w = (
            _bd_rowpad(W0[:d]), _bd_rowpad(W0[d : 2 * d]), _bd_rowpad(W0[2 * d :]),
            _tile8(b0), _bd(W1), _tile8(b1),
            _bd_colpad(W2), _tile8(jnp.pad(b2, (0, _P - b2.shape[0]))),
        )
        eA = _edge_mlp3(xiA.reshape(E8h, 128), xjA.reshape(E8h, 128), eaA,
                        *r1w, ea_off=ea_offs[0])
        eB = _edge_mlp3(xiB.reshape(E8h, 128), xjB.reshape(E8h, 128), eaB,
                        *r1w, ea_off=ea_offs[1])
        pA = _sc_scatter_add(eA.reshape(Eh, _P), dstA, zeros)
        pB = _sc_scatter_add(eB.reshape(Eh, _P), dstB, zeros)
        (V0, c0), (V1, c1), (V2, c2) = layer["O"]
        x8 = _node_mlp3(
            x8, pA.reshape(_NC * n8, 128), pB.reshape(_NC * n8, 128),
            _bd_rowpad(V0[:d]), _bd_rowpad(V0[d:]),
            _tile8(c0), _bd(V1), _tile8(c1),
            _bd_colpad(V2), _tile8(jnp.pad(c2, (0, _P - c2.shape[0]))),
        )
        eaA, eaB = eA, eB
        ea_offs = (0, 0)
        d = W2.shape[1]

    (W0, b0), (W1, b1), (W2, b2) = params["R2"]
    x16 = x8.reshape(_NPAD, _P)
    xiA, xjA = _sc_gather2(x16, dstA, srcA)
    xiB, xjB = _sc_gather2(x16, dstB, srcB)
    r2w = (
        _bd_rowpad(W0[:d]), _bd_rowpad(W0[d : 2 * d]), _bd_rowpad(W0[2 * d :]),
        _tile8(b0), _bd(W1), _tile8(b1), _bd(W2), _tile8(b2),
    )
    oA = _edge_mlp3(xiA.reshape(E8h, 128), xjA.reshape(E8h, 128), eaA,
                    *r2w, final=True, ea_off=ea_offs[0], out_rows=E8)
    oB = _edge_mlp3(xiB.reshape(E8h, 128), xjB.reshape(E8h, 128), eaB,
                    *r2w, final=True, ea_off=ea_offs[1],
                    out_rows=E8, out_off=hoff, out_alias=oA)
    return oB.reshape(E, 1)
